# baseline (speedup 1.0000x reference)
import functools

import jax
import jax.numpy as jnp
from jax import lax
from jax.experimental import pallas as pl
from jax.experimental.pallas import tpu as pltpu
from jax.experimental.pallas import tpu_sc as plsc

_NC = 2
_NS = 16
_NW = _NC * _NS
_LANES = 16
_CHUNK = 128
_TILE = 128

_SP_C0 = 0.693147186409334
_SP_C1 = 0.1249997313784969
_SP_C2 = -5.206379217398428e-03
_SP_C3 = 3.4224919293833467e-04
_SP_C4 = -2.109280949471386e-05


@functools.lru_cache(maxsize=None)
def _make_sc_partials(B, n_body):
    per_w = B // _NW
    n_chunk = per_w // _CHUNK
    n_vec = per_w // _LANES
    mesh = plsc.VectorSubcoreMesh(core_axis_name="c", subcore_axis_name="s")

    @functools.partial(
        pl.kernel,
        out_type=jax.ShapeDtypeStruct((_NW, _LANES), jnp.float32),
        mesh=mesh,
        scratch_types=[
            pltpu.VMEM((n_chunk, _CHUNK), jnp.int32),
            pltpu.VMEM((n_chunk, _CHUNK), jnp.int32),
            pltpu.VMEM((n_chunk, _CHUNK), jnp.int32),
            pltpu.VMEM((n_chunk, _CHUNK), jnp.int32),
            pltpu.VMEM((per_w,), jnp.int32),
            pltpu.VMEM((per_w,), jnp.float32),
            pltpu.VMEM((per_w,), jnp.float32),
            pltpu.VMEM((per_w,), jnp.float32),
            pltpu.VMEM((per_w,), jnp.float32),
            pltpu.VMEM((per_w,), jnp.float32),
            pltpu.VMEM((_LANES,), jnp.float32),
            pltpu.SemaphoreType.DMA,
        ],
    )
    def sc_partials(o_hbm, ridx_hbm, i0_hbm, i1_hbm, it0_hbm, it1_hbm,
                    body_hbm, tail_hbm, out_hbm,
                    i0_v, i1_v, it0_v, it1_v, ridx_v,
                    c0_v, c1_v, t0_v, t1_v, o_v, part_v, sem):
        wid = lax.axis_index("s") * _NC + lax.axis_index("c")
        base = wid * per_w
        sl_w = pl.ds(base, per_w)
        pltpu.sync_copy(i0_hbm.at[wid], i0_v)
        pltpu.sync_copy(i1_hbm.at[wid], i1_v)
        pltpu.sync_copy(it0_hbm.at[wid], it0_v)
        pltpu.sync_copy(it1_hbm.at[wid], it1_v)
        copies = []
        for k in range(n_chunk):
            sl = pl.ds(k * _CHUNK, _CHUNK)
            copies.append(pltpu.async_copy(
                body_hbm.at[i0_v.at[k]], c0_v.at[sl], sem))
            copies.append(pltpu.async_copy(
                body_hbm.at[i1_v.at[k]], c1_v.at[sl], sem))
            copies.append(pltpu.async_copy(
                tail_hbm.at[it0_v.at[k]], t0_v.at[sl], sem))
            copies.append(pltpu.async_copy(
                tail_hbm.at[it1_v.at[k]], t1_v.at[sl], sem))
        pltpu.sync_copy(o_hbm.at[sl_w], o_v)
        pltpu.sync_copy(ridx_hbm.at[sl_w], ridx_v)
        for c in copies:
            c.wait()

        def body(i, acc):
            sl = pl.ds(i * _LANES, _LANES)
            o = o_v[sl]
            in_tail = ridx_v[sl] >= n_body
            c0 = jnp.where(in_tail, t0_v[sl], c0_v[sl])
            c1 = jnp.where(in_tail, t1_v[sl], c1_v[sl])
            x = 2.0 * o - 1.0
            u = x * x
            sp = 0.5 * x + (_SP_C0 + u * (_SP_C1 + u * (
                _SP_C2 + u * (_SP_C3 + u * _SP_C4))))
            return acc + (sp * (c0 + c1) - x * c1)

        acc = lax.fori_loop(0, n_vec, body, jnp.zeros((_LANES,), jnp.float32))
        part_v[...] = acc
        pltpu.sync_copy(part_v, out_hbm.at[wid])

    return sc_partials


@functools.lru_cache(maxsize=None)
def _make_reduce(B):
    def body(p_ref, o_ref):
        o_ref[0, 0] = jnp.sum(p_ref[...]) * (1.0 / B)

    return pl.pallas_call(
        body,
        out_shape=jax.ShapeDtypeStruct((1, 1), jnp.float32),
        in_specs=[pl.BlockSpec(memory_space=pltpu.VMEM)],
        out_specs=pl.BlockSpec(memory_space=pltpu.SMEM),
    )


def kernel(outputs, patch_index, confidence):
    B = outputs.shape[0]
    per_w = B // _NW
    n_rows = confidence.shape[0]
    n_full = n_rows // _TILE
    n_body = n_full * _TILE
    o_flat = outputs.reshape((B,))

    body = (confidence[:n_body]
            .reshape(n_full, _TILE, 2)
            .transpose(0, 2, 1)
            .reshape(2 * n_body))
    tail2 = confidence[n_body:].transpose(1, 0)
    tail = jnp.pad(tail2, ((0, 0), (0, _TILE - tail2.shape[1]))).reshape(-1)

    tailm = patch_index >= n_body
    i0 = jnp.where(tailm, 0, patch_index + (patch_index & -_TILE))
    i1 = i0 + _TILE
    it0 = jnp.where(tailm, patch_index - n_body, 0)
    it1 = it0 + _TILE
    shp = (_NW, per_w // _CHUNK, _CHUNK)
    partials = _make_sc_partials(B, n_body)(
        o_flat, patch_index, i0.reshape(shp), i1.reshape(shp),
        it0.reshape(shp), it1.reshape(shp), body, tail)
    return _make_reduce(B)(partials)[0, 0]


# speedup vs baseline: 1.0279x; 1.0279x over previous
import functools

import jax
import jax.numpy as jnp
from jax import lax
from jax.experimental import pallas as pl
from jax.experimental.pallas import tpu as pltpu
from jax.experimental.pallas import tpu_sc as plsc

_NC = 2
_NS = 16
_NW = _NC * _NS
_LANES = 16
_CHUNK = 128
_TILE = 128

_SP_C0 = 0.693147186409334
_SP_C1 = 0.1249997313784969
_SP_C2 = -5.206379217398428e-03
_SP_C3 = 3.4224919293833467e-04
_SP_C4 = -2.109280949471386e-05


@functools.lru_cache(maxsize=None)
def _make_sc_partials(B, n_body):
    per_w = B // _NW
    n_chunk = per_w // _CHUNK
    n_vec = per_w // _LANES
    mesh = plsc.VectorSubcoreMesh(core_axis_name="c", subcore_axis_name="s")

    @functools.partial(
        pl.kernel,
        out_type=jax.ShapeDtypeStruct((_NW, _LANES), jnp.float32),
        mesh=mesh,
        scratch_types=[
            pltpu.VMEM((n_chunk, _CHUNK), jnp.int32),
            pltpu.VMEM((n_chunk, _CHUNK), jnp.int32),
            pltpu.VMEM((n_chunk, _CHUNK), jnp.int32),
            pltpu.VMEM((per_w,), jnp.int32),
            pltpu.VMEM((per_w,), jnp.float32),
            pltpu.VMEM((per_w,), jnp.float32),
            pltpu.VMEM((per_w,), jnp.float32),
            pltpu.VMEM((per_w,), jnp.float32),
            pltpu.VMEM((per_w,), jnp.float32),
            pltpu.VMEM((_LANES,), jnp.float32),
            pltpu.SemaphoreType.DMA,
        ],
    )
    def sc_partials(o_hbm, ridx_hbm, ib_hbm, it0_hbm, it1_hbm,
                    c0_hbm, c1_hbm, tail_hbm, out_hbm,
                    ib_v, it0_v, it1_v, ridx_v,
                    c0_v, c1_v, t0_v, t1_v, o_v, part_v, sem):
        wid = lax.axis_index("s") * _NC + lax.axis_index("c")
        base = wid * per_w
        sl_w = pl.ds(base, per_w)
        pltpu.sync_copy(ib_hbm.at[wid], ib_v)
        pltpu.sync_copy(it0_hbm.at[wid], it0_v)
        pltpu.sync_copy(it1_hbm.at[wid], it1_v)
        copies = []
        for k in range(n_chunk):
            sl = pl.ds(k * _CHUNK, _CHUNK)
            copies.append(pltpu.async_copy(
                c0_hbm.at[ib_v.at[k]], c0_v.at[sl], sem))
            copies.append(pltpu.async_copy(
                c1_hbm.at[ib_v.at[k]], c1_v.at[sl], sem))
            copies.append(pltpu.async_copy(
                tail_hbm.at[it0_v.at[k]], t0_v.at[sl], sem))
            copies.append(pltpu.async_copy(
                tail_hbm.at[it1_v.at[k]], t1_v.at[sl], sem))
        pltpu.sync_copy(o_hbm.at[sl_w], o_v)
        pltpu.sync_copy(ridx_hbm.at[sl_w], ridx_v)
        for c in copies:
            c.wait()

        def body(i, acc):
            sl = pl.ds(i * _LANES, _LANES)
            o = o_v[sl]
            in_tail = ridx_v[sl] >= n_body
            c0 = jnp.where(in_tail, t0_v[sl], c0_v[sl])
            c1 = jnp.where(in_tail, t1_v[sl], c1_v[sl])
            x = 2.0 * o - 1.0
            u = x * x
            sp = 0.5 * x + (_SP_C0 + u * (_SP_C1 + u * (
                _SP_C2 + u * (_SP_C3 + u * _SP_C4))))
            return acc + (sp * (c0 + c1) - x * c1)

        acc = lax.fori_loop(0, n_vec, body, jnp.zeros((_LANES,), jnp.float32))
        part_v[...] = acc
        pltpu.sync_copy(part_v, out_hbm.at[wid])

    return sc_partials


@functools.lru_cache(maxsize=None)
def _make_reduce(B):
    def body(p_ref, o_ref):
        o_ref[0, 0] = jnp.sum(p_ref[...]) * (1.0 / B)

    return pl.pallas_call(
        body,
        out_shape=jax.ShapeDtypeStruct((1, 1), jnp.float32),
        in_specs=[pl.BlockSpec(memory_space=pltpu.VMEM)],
        out_specs=pl.BlockSpec(memory_space=pltpu.SMEM),
    )


def kernel(outputs, patch_index, confidence):
    B = outputs.shape[0]
    per_w = B // _NW
    n_rows = confidence.shape[0]
    n_full = n_rows // _TILE
    n_body = n_full * _TILE
    o_flat = outputs.reshape((B,))

    blocks = confidence[:n_body].reshape(n_full, _TILE, 2)
    c0 = blocks[:, :, 0].reshape(-1)
    c1 = blocks[:, :, 1].reshape(-1)
    tail2 = confidence[n_body:].transpose(1, 0)
    tail = jnp.pad(tail2, ((0, 0), (0, _TILE - tail2.shape[1]))).reshape(-1)

    tailm = patch_index >= n_body
    ib = jnp.where(tailm, 0, patch_index)
    it0 = jnp.where(tailm, patch_index - n_body, 0)
    it1 = it0 + _TILE
    shp = (_NW, per_w // _CHUNK, _CHUNK)
    partials = _make_sc_partials(B, n_body)(
        o_flat, patch_index, ib.reshape(shp),
        it0.reshape(shp), it1.reshape(shp), c0, c1, tail)
    return _make_reduce(B)(partials)[0, 0]


# speedup vs baseline: 2.6458x; 2.5741x over previous
import functools

import jax
import jax.numpy as jnp
from jax import lax
from jax.experimental import pallas as pl
from jax.experimental.pallas import tpu as pltpu
from jax.experimental.pallas import tpu_sc as plsc

_NC = 2
_NS = 16
_NW = _NC * _NS
_LANES = 16
_CHUNK = 128
_TILE = 128

_SP_C0 = 0.693147186409334
_SP_C1 = 0.1249997313784969
_SP_C2 = -5.206379217398428e-03
_SP_C3 = 3.4224919293833467e-04
_SP_C4 = -2.109280949471386e-05


@functools.lru_cache(maxsize=None)
def _make_sc_partials(B):
    per_w = B // _NW
    n_chunk = per_w // _CHUNK
    n_vec = per_w // _LANES
    mesh = plsc.VectorSubcoreMesh(core_axis_name="c", subcore_axis_name="s")

    @functools.partial(
        pl.kernel,
        out_type=jax.ShapeDtypeStruct((_NW, _LANES), jnp.float32),
        mesh=mesh,
        scratch_types=[
            pltpu.VMEM((n_chunk, _CHUNK), jnp.int32),
            pltpu.VMEM((per_w,), jnp.float32),
            pltpu.VMEM((per_w,), jnp.float32),
            pltpu.VMEM((per_w,), jnp.float32),
            pltpu.VMEM((_LANES,), jnp.float32),
            pltpu.SemaphoreType.DMA,
        ],
    )
    def sc_partials(o_hbm, idx_hbm, c0_hbm, c1_hbm, out_hbm,
                    idx_v, c0_v, c1_v, o_v, part_v, sem):
        wid = lax.axis_index("s") * _NC + lax.axis_index("c")
        base = wid * per_w
        pltpu.sync_copy(idx_hbm.at[wid], idx_v)
        copies = []
        for k in range(n_chunk):
            sl = pl.ds(k * _CHUNK, _CHUNK)
            copies.append(pltpu.async_copy(
                c0_hbm.at[idx_v.at[k]], c0_v.at[sl], sem))
            copies.append(pltpu.async_copy(
                c1_hbm.at[idx_v.at[k]], c1_v.at[sl], sem))
        pltpu.sync_copy(o_hbm.at[pl.ds(base, per_w)], o_v)
        for c in copies:
            c.wait()

        def body(i, acc):
            sl = pl.ds(i * _LANES, _LANES)
            o = o_v[sl]
            c0 = c0_v[sl]
            c1 = c1_v[sl]
            x = 2.0 * o - 1.0
            u = x * x
            sp = 0.5 * x + (_SP_C0 + u * (_SP_C1 + u * (
                _SP_C2 + u * (_SP_C3 + u * _SP_C4))))
            return acc + (sp * (c0 + c1) - x * c1)

        acc = lax.fori_loop(0, n_vec, body, jnp.zeros((_LANES,), jnp.float32))
        part_v[...] = acc
        pltpu.sync_copy(part_v, out_hbm.at[wid])

    return sc_partials


@functools.lru_cache(maxsize=None)
def _make_reduce(B):
    def body(p_ref, o_ref):
        o_ref[0, 0] = jnp.sum(p_ref[...]) * (1.0 / B)

    return pl.pallas_call(
        body,
        out_shape=jax.ShapeDtypeStruct((1, 1), jnp.float32),
        in_specs=[pl.BlockSpec(memory_space=pltpu.VMEM)],
        out_specs=pl.BlockSpec(memory_space=pltpu.SMEM),
    )


def kernel(outputs, patch_index, confidence):
    B = outputs.shape[0]
    per_w = B // _NW
    n_rows = confidence.shape[0]
    n_full = n_rows // _TILE
    n_body = n_full * _TILE
    o_flat = outputs.reshape((B,))

    blocks = confidence[:n_body].reshape(n_full, _TILE, 2)
    n_pad = n_body + _TILE - n_rows
    c0 = jnp.concatenate([blocks[:, :, 0].reshape(-1),
                          jnp.pad(confidence[n_body:, 0], (0, n_pad))])
    c1 = jnp.concatenate([blocks[:, :, 1].reshape(-1),
                          jnp.pad(confidence[n_body:, 1], (0, n_pad))])

    idx = patch_index.reshape((_NW, per_w // _CHUNK, _CHUNK))
    partials = _make_sc_partials(B)(o_flat, idx, c0, c1)
    return _make_reduce(B)(partials)[0, 0]


# speedup vs baseline: 2.8915x; 1.0928x over previous
import functools

import jax
import jax.numpy as jnp
from jax import lax
from jax.experimental import pallas as pl
from jax.experimental.pallas import tpu as pltpu
from jax.experimental.pallas import tpu_sc as plsc

_NC = 2
_NS = 16
_NW = _NC * _NS
_LANES = 16
_CHUNK = 128
_TILE = 128

_SP_C0 = 0.693147186409334
_SP_C1 = 0.1249997313784969
_SP_C2 = -5.206379217398428e-03
_SP_C3 = 3.4224919293833467e-04
_SP_C4 = -2.109280949471386e-05


@functools.lru_cache(maxsize=None)
def _make_sc_partials(B):
    per_w = B // _NW
    n_chunk = per_w // _CHUNK
    n_vec = per_w // _LANES
    mesh = plsc.VectorSubcoreMesh(core_axis_name="c", subcore_axis_name="s")

    @functools.partial(
        pl.kernel,
        out_type=jax.ShapeDtypeStruct((_NW, _LANES), jnp.float32),
        mesh=mesh,
        scratch_types=[
            pltpu.VMEM((n_chunk, _CHUNK), jnp.int32),
            pltpu.VMEM((per_w,), jnp.float32),
            pltpu.VMEM((per_w,), jnp.float32),
            pltpu.VMEM((_LANES,), jnp.float32),
            pltpu.SemaphoreType.DMA,
        ],
    )
    def sc_partials(o_hbm, idx_hbm, c1_hbm, out_hbm,
                    idx_v, c1_v, o_v, part_v, sem):
        wid = lax.axis_index("s") * _NC + lax.axis_index("c")
        base = wid * per_w
        pltpu.sync_copy(idx_hbm.at[wid], idx_v)
        copies = []
        for k in range(n_chunk):
            sl = pl.ds(k * _CHUNK, _CHUNK)
            copies.append(pltpu.async_copy(
                c1_hbm.at[idx_v.at[k]], c1_v.at[sl], sem))
        pltpu.sync_copy(o_hbm.at[pl.ds(base, per_w)], o_v)
        for c in copies:
            c.wait()

        def body(i, acc):
            sl = pl.ds(i * _LANES, _LANES)
            o = o_v[sl]
            c1 = c1_v[sl]
            x = 2.0 * o - 1.0
            u = x * x
            sp = 0.5 * x + (_SP_C0 + u * (_SP_C1 + u * (
                _SP_C2 + u * (_SP_C3 + u * _SP_C4))))
            return acc + (sp - x * c1)

        acc = lax.fori_loop(0, n_vec, body, jnp.zeros((_LANES,), jnp.float32))
        part_v[...] = acc
        pltpu.sync_copy(part_v, out_hbm.at[wid])

    return sc_partials


@functools.lru_cache(maxsize=None)
def _make_reduce(B):
    def body(p_ref, o_ref):
        o_ref[0, 0] = jnp.sum(p_ref[...]) * (1.0 / B)

    return pl.pallas_call(
        body,
        out_shape=jax.ShapeDtypeStruct((1, 1), jnp.float32),
        in_specs=[pl.BlockSpec(memory_space=pltpu.VMEM)],
        out_specs=pl.BlockSpec(memory_space=pltpu.SMEM),
    )


def kernel(outputs, patch_index, confidence):
    B = outputs.shape[0]
    per_w = B // _NW
    n_rows = confidence.shape[0]
    n_full = n_rows // _TILE
    n_body = n_full * _TILE
    o_flat = outputs.reshape((B,))

    blocks = confidence[:n_body].reshape(n_full, _TILE, 2)
    n_pad = n_body + _TILE - n_rows
    c1 = jnp.concatenate([blocks[:, :, 1].reshape(-1),
                          jnp.pad(confidence[n_body:, 1], (0, n_pad))])

    idx = patch_index.reshape((_NW, per_w // _CHUNK, _CHUNK))
    partials = _make_sc_partials(B)(o_flat, idx, c1)
    return _make_reduce(B)(partials)[0, 0]
